# Initial kernel scaffold; baseline (speedup 1.0000x reference)
import jax
import jax.numpy as jnp
from jax.experimental import pallas as pl


def kernel(x, edge_index, batch, cond, W_gcn, b_gcn, W_proj, b_proj, W_fc, b_fc):
    raise NotImplementedError("write your pallas kernel here")


# speedup vs baseline: 1.0920x; 1.0920x over previous
import functools

import jax
import jax.numpy as jnp
from jax.experimental import pallas as pl


N_NODES = 10000
DIM = 256
NUM_GRAPHS = 16
ROW_BLK = 1024
N_PAD = 10240


def _matmul_body(x_ref, w_ref, o_ref):
    o_ref[...] = jnp.dot(x_ref[...], w_ref[...],
                         preferred_element_type=jnp.float32)


def _matmul(x_pad, w):
    grid = (N_PAD // ROW_BLK,)
    return pl.pallas_call(
        _matmul_body,
        grid=grid,
        in_specs=[
            pl.BlockSpec((ROW_BLK, DIM), lambda i: (i, 0)),
            pl.BlockSpec((DIM, DIM), lambda i: (0, 0)),
        ],
        out_specs=pl.BlockSpec((ROW_BLK, DIM), lambda i: (i, 0)),
        out_shape=jax.ShapeDtypeStruct((N_PAD, DIM), jnp.float32),
    )(x_pad, w)


def _spectral_normalize(W, n_iter=5):
    u = jnp.ones((W.shape[0],), dtype=W.dtype) / jnp.sqrt(W.shape[0])
    v = jnp.ones((W.shape[1],), dtype=W.dtype) / jnp.sqrt(W.shape[1])
    for _ in range(n_iter):
        v = W.T @ u
        v = v / (jnp.linalg.norm(v) + 1e-12)
        u = W @ v
        u = u / (jnp.linalg.norm(u) + 1e-12)
    sigma = u @ (W @ v)
    return W / sigma


def kernel(x, edge_index, batch, cond, W_gcn, b_gcn, W_proj, b_proj, W_fc, b_fc):
    n = x.shape[0]
    x_pad = jnp.pad(x, ((0, N_PAD - n), (0, 0)))
    h = _matmul(x_pad, W_gcn)[:n]

    src = edge_index[0].astype(jnp.int32)
    dst = edge_index[1].astype(jnp.int32)
    deg = jnp.ones((n,), jnp.float32).at[dst].add(1.0)
    dis = jax.lax.rsqrt(deg)
    norm = dis[src] * dis[dst]
    msg = h[src] * norm[:, None]
    out = (h * (1.0 / deg)[:, None]).at[dst].add(msg) + b_gcn
    out = jax.nn.leaky_relu(out, 0.2)
    pooled = jax.ops.segment_max(out, batch, num_segments=NUM_GRAPHS)

    Wp = _spectral_normalize(W_proj)
    c = cond @ Wp.T + b_proj
    proj = jnp.sum(pooled * c, axis=1, keepdims=True)
    Wf = _spectral_normalize(W_fc)
    y = pooled @ Wf.T + b_fc + proj
    return y


# speedup vs baseline: 6.2083x; 5.6855x over previous
import dataclasses
import functools

import jax
import jax.numpy as jnp
from jax import lax
from jax.experimental import pallas as pl
from jax.experimental.pallas import tpu as pltpu
from jax.experimental.pallas import tpu_sc as plsc


N_NODES = 10000
DIM = 256
NUM_GRAPHS = 16
N_PAD = 10240
ROW_BLK = 1024
N_BLKS = N_PAD // ROW_BLK

E_EDGES = 160000
E_PAD = 161280
PAD_IDX = N_PAD - 1

GATHER_BLK = 48
EDGES_PER_TILE = E_PAD // 32
AGG_ITERS = EDGES_PER_TILE // GATHER_BLK
DEG_GROUPS = EDGES_PER_TILE // 16

_SC_MESH = plsc.VectorSubcoreMesh(core_axis_name="c", subcore_axis_name="s")

_SC_PARAMS = pltpu.CompilerParams()
if "needs_layout_passes" in pltpu.CompilerParams.__dataclass_fields__:
    _SC_PARAMS = dataclasses.replace(_SC_PARAMS, needs_layout_passes=False)

OWN_ROWS = N_PAD // 32
CAP = 8192
SCAN_CHUNK = 8064
GATHER_BLK = 32
E_PAD = 161280
PAD_IDX = N_PAD - 1


def _scan_body(src_hbm, dst_hbm, csrc_o, cdst_o, kcnt_o, deg_o,
               schunk, dchunk, csrcb, cdstb, offb, kvb, degb, degwide):
    c = lax.axis_index("c")
    s = lax.axis_index("s")
    w = c * 16 + s
    base = w * OWN_ROWS

    @pl.loop(0, OWN_ROWS + 8)
    def _(j):
        degb[j] = 0.0
    offb[0] = 0

    for chunk in range(E_PAD // SCAN_CHUNK):
        pltpu.sync_copy(src_hbm.at[pl.ds(chunk * SCAN_CHUNK, SCAN_CHUNK)],
                        schunk)
        pltpu.sync_copy(dst_hbm.at[pl.ds(chunk * SCAN_CHUNK, SCAN_CHUNK)],
                        dchunk)

        @pl.loop(0, SCAN_CHUNK // 16)
        def _(g):
            s16 = schunk[pl.ds(g * 16, 16)]
            d16 = dchunk[pl.ds(g * 16, 16)]
            ld16 = d16 - base
            own = (ld16 >= 0) & (ld16 < OWN_ROWS)
            owni = jnp.where(own, 1, 0).astype(jnp.int32)
            packed = s16 * 512 + (ld16 & 511)
            _, psort = plsc.sort_key_val(1 - owni, packed)
            off = offb[0]
            cdstb[pl.ds(off, 16)] = psort
            offb[0] = off + lax.reduce_sum(owni, axes=(0,))

    k = offb[0]

    kp = ((k >> 6) + 1) << 6
    for j in range(4):
        cdstb[pl.ds(k + j * 16, 16)] = jnp.full(
            (16,), PAD_IDX * 512 + OWN_ROWS, jnp.int32)

    @pl.loop(0, kp >> 4)
    def _(g):
        v = cdstb[pl.ds(g * 16, 16)]
        csrcb[pl.ds(g * 16, 16)] = lax.shift_right_logical(v, 9)
        cdstb[pl.ds(g * 16, 16)] = v & 511

    @pl.loop(0, kp >> 4)
    def _(g):
        dvec = cdstb[pl.ds(g * 16, 16)]
        for rr in range(16):
            d = dvec[rr]
            degb[d] = degb[d] + 1.0

    kvb[pl.ds(0, 16)] = jnp.ones((16,), jnp.int32) * kp
    pltpu.sync_copy(kvb, kcnt_o.at[pl.ds(w * 16, 16)])
    pltpu.sync_copy(csrcb, csrc_o.at[w])
    pltpu.sync_copy(cdstb, cdst_o.at[w])

    ones16 = jnp.ones((16,), jnp.float32)

    @pl.loop(0, OWN_ROWS)
    def _(r):
        spl = ones16 * degb[r]
        for t in range(8):
            degwide.at[r][pl.ds(t * 16, 16)] = spl

    pltpu.sync_copy(degwide, deg_o.at[pl.ds(base, OWN_ROWS)])


_sc_scan = pl.kernel(
    _scan_body,
    out_type=(
        jax.ShapeDtypeStruct((32, CAP), jnp.int32),
        jax.ShapeDtypeStruct((32, CAP), jnp.int32),
        jax.ShapeDtypeStruct((512,), jnp.int32),
        jax.ShapeDtypeStruct((N_PAD, 128), jnp.float32),
    ),
    mesh=_SC_MESH,
    compiler_params=_SC_PARAMS,
    scratch_types=[
        pltpu.VMEM((SCAN_CHUNK,), jnp.int32),
        pltpu.VMEM((SCAN_CHUNK,), jnp.int32),
        pltpu.VMEM((CAP,), jnp.int32),
        pltpu.VMEM((CAP,), jnp.int32),
        pltpu.SMEM((8,), jnp.int32),
        pltpu.VMEM((16,), jnp.int32),
        pltpu.SMEM((OWN_ROWS + 8,), jnp.float32),
        pltpu.VMEM((OWN_ROWS, 128), jnp.float32),
    ],
)


def _agg_body(csrc_h, cdst_h, kcnt_h, hs_hbm, out_hbm,
              csrcb, cdstb, kvb, acc, gbuf0, gbuf1, sem0, sem1):
    c = lax.axis_index("c")
    s = lax.axis_index("s")
    w = c * 16 + s
    base = w * OWN_ROWS

    pltpu.sync_copy(csrc_h.at[w], csrcb)
    pltpu.sync_copy(cdst_h.at[w], cdstb)
    pltpu.sync_copy(kcnt_h.at[pl.ds(w * 16, 16)], kvb)
    pltpu.sync_copy(hs_hbm.at[pl.ds(base, OWN_ROWS)],
                    acc.at[pl.ds(0, OWN_ROWS)])

    ng = kvb[pl.ds(0, 16)][0] >> 5

    def gstart(i, gbuf, sem):
        pltpu.make_async_copy(
            hs_hbm.at[csrcb.at[pl.ds(i * GATHER_BLK, GATHER_BLK)]],
            gbuf, sem).start()

    def gwait(i, gbuf, sem):
        pltpu.make_async_copy(
            hs_hbm.at[csrcb.at[pl.ds(i * GATHER_BLK, GATHER_BLK)]],
            gbuf, sem).wait()

    def rmw(i, gbuf):
        for h in range(GATHER_BLK // 16):
            dvec = cdstb[pl.ds(i * GATHER_BLK + h * 16, 16)]
            for rr in range(16):
                ld = dvec[rr]
                row = acc.at[ld]
                gr = gbuf.at[h * 16 + rr]
                for cc in range(DIM // 16):
                    plsc.addupdate(row.at[pl.ds(cc * 16, 16)],
                                   gr[pl.ds(cc * 16, 16)])

    gstart(0, gbuf0, sem0)

    @pl.loop(0, ng, step=2)
    def _(i):
        gstart(i + 1, gbuf1, sem1)
        gwait(i, gbuf0, sem0)
        rmw(i, gbuf0)

        @pl.when(i + 2 < ng)
        def _():
            gstart(i + 2, gbuf0, sem0)

        gwait(i + 1, gbuf1, sem1)
        rmw(i + 1, gbuf1)

    pltpu.sync_copy(acc.at[pl.ds(0, OWN_ROWS)],
                    out_hbm.at[pl.ds(base, OWN_ROWS)])


_sc_aggregate = pl.kernel(
    _agg_body,
    out_type=jax.ShapeDtypeStruct((N_PAD, DIM), jnp.float32),
    mesh=_SC_MESH,
    scratch_types=[
        pltpu.VMEM((CAP,), jnp.int32),
        pltpu.VMEM((CAP,), jnp.int32),
        pltpu.VMEM((16,), jnp.int32),
        pltpu.VMEM((OWN_ROWS + 8, DIM), jnp.float32),
        pltpu.VMEM((GATHER_BLK, DIM), jnp.float32),
        pltpu.VMEM((GATHER_BLK, DIM), jnp.float32),
        pltpu.SemaphoreType.DMA,
        pltpu.SemaphoreType.DMA,
    ],
)


def _mm_body(x_ref, w_ref, deg_ref, o_ref):
    dis = lax.rsqrt(deg_ref[:, 0:1] + 1.0)
    o_ref[...] = jnp.dot(x_ref[...], w_ref[...],
                         preferred_element_type=jnp.float32) * dis


def _tc_matmul_scale(x_pad, w, deg):
    return pl.pallas_call(
        _mm_body,
        grid=(N_BLKS,),
        in_specs=[
            pl.BlockSpec((ROW_BLK, DIM), lambda i: (i, 0)),
            pl.BlockSpec((DIM, DIM), lambda i: (0, 0)),
            pl.BlockSpec((ROW_BLK, 128), lambda i: (i, 0)),
        ],
        out_specs=pl.BlockSpec((ROW_BLK, DIM), lambda i: (i, 0)),
        out_shape=jax.ShapeDtypeStruct((N_PAD, DIM), jnp.float32),
    )(x_pad, w, deg)


def _power_iter(W, n_rows, n_cols):
    u = jnp.ones((1, n_rows), jnp.float32) / jnp.sqrt(float(n_rows))
    v = jnp.ones((1, n_cols), jnp.float32) / jnp.sqrt(float(n_cols))
    for _ in range(5):
        v = lax.dot_general(u, W, (((1,), (0,)), ((), ())),
                            preferred_element_type=jnp.float32)
        v = v / (jnp.sqrt(jnp.sum(v * v)) + 1e-12)
        u = lax.dot_general(v, W, (((1,), (1,)), ((), ())),
                            preferred_element_type=jnp.float32)
        u = u / (jnp.sqrt(jnp.sum(u * u)) + 1e-12)
    wv = lax.dot_general(v, W, (((1,), (1,)), ((), ())),
                         preferred_element_type=jnp.float32)
    return jnp.sum(u * wv)


def _tail_body(acc_ref, deg_ref, bat_ref, bg_ref, cond_ref, wp_ref, bp_ref,
               wf_ref, bf_ref, y_ref, pooled):
    i = pl.program_id(0)

    @pl.when(i == 0)
    def _():
        pooled[...] = jnp.full((NUM_GRAPHS, DIM), -jnp.inf, jnp.float32)

    dis = lax.rsqrt(deg_ref[:, 0:1] + 1.0)
    act = acc_ref[...] * dis + bg_ref[...]
    act = jnp.where(act >= 0.0, act, 0.2 * act)
    bid = bat_ref[...]
    for b in range(NUM_GRAPHS):
        m = bid == b
        mx = jnp.max(jnp.where(m, act, -jnp.inf), axis=0, keepdims=True)
        pooled[b:b + 1, :] = jnp.maximum(pooled[b:b + 1, :], mx)

    @pl.when(i == N_BLKS - 1)
    def _():
        pool = pooled[...]
        Wp = wp_ref[...]
        sig_p = _power_iter(Wp, DIM, DIM)
        c = lax.dot_general(cond_ref[...], Wp, (((1,), (1,)), ((), ())),
                            preferred_element_type=jnp.float32) / sig_p
        c = c + bp_ref[...]
        proj = jnp.sum(pool * c, axis=1, keepdims=True)
        Wf = wf_ref[...]
        sig_f = _power_iter(Wf, 1, DIM)
        y = lax.dot_general(pool, Wf, (((1,), (1,)), ((), ())),
                            preferred_element_type=jnp.float32) / sig_f
        y_ref[...] = y + bf_ref[...] + proj


def _tc_tail(acc, deg, batch2d, b_gcn, cond, W_proj, b_proj, W_fc, b_fc):
    return pl.pallas_call(
        _tail_body,
        grid=(N_BLKS,),
        in_specs=[
            pl.BlockSpec((ROW_BLK, DIM), lambda i: (i, 0)),
            pl.BlockSpec((ROW_BLK, 128), lambda i: (i, 0)),
            pl.BlockSpec((ROW_BLK, 1), lambda i: (i, 0)),
            pl.BlockSpec((1, DIM), lambda i: (0, 0)),
            pl.BlockSpec((NUM_GRAPHS, DIM), lambda i: (0, 0)),
            pl.BlockSpec((DIM, DIM), lambda i: (0, 0)),
            pl.BlockSpec((1, DIM), lambda i: (0, 0)),
            pl.BlockSpec((1, DIM), lambda i: (0, 0)),
            pl.BlockSpec((1, 1), lambda i: (0, 0)),
        ],
        out_specs=pl.BlockSpec((NUM_GRAPHS, 1), lambda i: (0, 0)),
        out_shape=jax.ShapeDtypeStruct((NUM_GRAPHS, 1), jnp.float32),
        scratch_shapes=[pltpu.VMEM((NUM_GRAPHS, DIM), jnp.float32)],
    )(acc, deg, batch2d, b_gcn, cond, W_proj, b_proj, W_fc, b_fc)


def kernel(x, edge_index, batch, cond, W_gcn, b_gcn, W_proj, b_proj, W_fc, b_fc):
    n = x.shape[0]
    src = edge_index[0].astype(jnp.int32)
    dst = edge_index[1].astype(jnp.int32)
    pad = jnp.full((E_PAD - E_EDGES,), PAD_IDX, jnp.int32)
    src_pad = jnp.concatenate([src, pad])
    dst_pad = jnp.concatenate([dst, pad])
    x_pad = jnp.pad(x, ((0, N_PAD - n), (0, 0)))
    batch2d = jnp.pad(batch.astype(jnp.int32), (0, N_PAD - n),
                      constant_values=NUM_GRAPHS).reshape(N_PAD, 1)

    csrc, cdst, kcnt, deg = _sc_scan(src_pad, dst_pad)
    hs = _tc_matmul_scale(x_pad, W_gcn, deg)
    acc = _sc_aggregate(csrc, cdst, kcnt, hs)
    y = _tc_tail(acc, deg, batch2d, b_gcn.reshape(1, DIM), cond,
                 W_proj, b_proj.reshape(1, DIM), W_fc, b_fc.reshape(1, 1))
    return y
